# baseline (speedup 1.0000x reference)
import functools

import jax
import jax.numpy as jnp
from jax import lax
from jax.experimental import pallas as pl
from jax.experimental.pallas import tpu as pltpu
from jax.experimental.pallas import tpu_sc as plsc

B = 4
S = 8192
D = 768
LANES = 16
NC = 2
NS = 16
NW = NC * NS
ROWS_PER_W = S // NW
K = 32
NCHUNK = ROWS_PER_W // K
DVEC = D // LANES
NSTEP = NCHUNK * B


def _sc_kernel(enc_hbm, tidx_hbm, pos_hbm, out_hbm, idx_v, t0_v, rows_v,
               buf_v, sem_g, sem_e, sem_o):
    wid = lax.axis_index("s") * NC + lax.axis_index("c")
    base = wid * ROWS_PER_W

    pltpu.sync_copy(tidx_hbm.at[pl.ds(0, LANES)], t0_v)
    pltpu.sync_copy(tidx_hbm.at[pl.ds(base, ROWS_PER_W)], idx_v)
    t0 = lax.gather(
        t0_v[...],
        jnp.zeros((LANES, 1), jnp.int32),
        dimension_numbers=lax.GatherDimensionNumbers(
            offset_dims=(), collapsed_slice_dims=(0,), start_index_map=(0,)),
        slice_sizes=(1,),
        mode=lax.GatherScatterMode.PROMISE_IN_BOUNDS)
    for j in range(ROWS_PER_W // LANES):
        sl = pl.ds(j * LANES, LANES)
        idx_v[sl] = idx_v[sl] - t0

    def issue_gather(ci, phase):
        pltpu.async_copy(
            pos_hbm.at[idx_v.at[pl.ds(ci * K, K)]], rows_v.at[phase], sem_g)

    def drain_gather(ci, phase):
        pltpu.make_async_copy(
            pos_hbm.at[idx_v.at[pl.ds(ci * K, K)]], rows_v.at[phase],
            sem_g).wait()

    def issue_enc(i):
        ci, b = i // B, i % B
        pltpu.async_copy(
            enc_hbm.at[b, pl.ds(base + ci * K, K)], buf_v.at[i % 3], sem_e)

    def drain_enc(slot):
        pltpu.make_async_copy(
            enc_hbm.at[0, pl.ds(0, K)], buf_v.at[slot], sem_e).wait()

    def issue_out(i, slot):
        ci, b = i // B, i % B
        pltpu.async_copy(
            buf_v.at[slot], out_hbm.at[b, pl.ds(base + ci * K, K)], sem_o)

    def drain_out(slot):
        pltpu.make_async_copy(
            buf_v.at[slot], out_hbm.at[0, pl.ds(0, K)], sem_o).wait()

    def add_rows(slot, phase):
        def row_body(r, c2):
            for c in range(DVEC):
                sl = pl.ds(c * LANES, LANES)
                plsc.addupdate(buf_v.at[slot, r, sl], rows_v[phase, r, sl])
            return c2
        lax.fori_loop(0, K, row_body, 0, unroll=2)

    def step(i, ci, b, phase, first_chunk, last_chunk):
        slot = i % 3
        nslot = (i + 1) % 3
        if b == 0:
            drain_gather(ci, phase)
            if not last_chunk:
                issue_gather(ci + 1, 1 - phase)
        if not (last_chunk and b == B - 1):
            if not (first_chunk and b < 2):
                drain_out(nslot)
            issue_enc(i + 1)
        drain_enc(slot)
        add_rows(slot, phase)
        issue_out(i, slot)

    issue_gather(0, 0)
    issue_enc(0)

    for b in range(B):
        step(b, 0, b, 0, True, False)

    def chunk_body(ci, carry):
        phase = ci % 2
        for b in range(B):
            step(ci * B + b, ci, b, phase, False, False)
        return carry

    lax.fori_loop(1, NCHUNK - 1, chunk_body, 0)

    lci = NCHUNK - 1
    for b in range(B):
        step(lci * B + b, lci, b, lci % 2, False, True)

    for s in range(3):
        drain_out(s)


@jax.jit
def _run(enc_inputs, tindex, pos_table):
    mesh = plsc.VectorSubcoreMesh(core_axis_name="c", subcore_axis_name="s")
    kfn = functools.partial(
        pl.kernel,
        mesh=mesh,
        out_type=jax.ShapeDtypeStruct((B, S, D), jnp.float32),
        scratch_types=[
            pltpu.VMEM((ROWS_PER_W,), jnp.int32),
            pltpu.VMEM((LANES,), jnp.int32),
            pltpu.VMEM((2, K, D), jnp.float32),
            pltpu.VMEM((3, K, D), jnp.float32),
            pltpu.SemaphoreType.DMA,
            pltpu.SemaphoreType.DMA,
            pltpu.SemaphoreType.DMA,
        ],
    )(_sc_kernel)
    return kfn(enc_inputs, tindex, pos_table)


def kernel(enc_inputs, tindex, pos_table):
    return _run(enc_inputs, tindex, pos_table)


# speedup vs baseline: 1.4921x; 1.4921x over previous
import functools

import jax
import jax.numpy as jnp
from jax import lax
from jax.experimental import pallas as pl
from jax.experimental.pallas import tpu as pltpu
from jax.experimental.pallas import tpu_sc as plsc

B = 4
S = 8192
D = 768
LANES = 16
NC = 2
NS = 16
NW = NC * NS
ROWS_PER_W = S // NW
K = 64
NCHUNK = ROWS_PER_W // K

BS = 512
NSB = S // BS


def _sc_gather(tidx_hbm, pos_hbm, rows_hbm, idx_v, t0_v, buf_v, sem_g,
               sem_o):
    wid = lax.axis_index("s") * NC + lax.axis_index("c")
    base = wid * ROWS_PER_W

    pltpu.sync_copy(tidx_hbm.at[pl.ds(0, LANES)], t0_v)
    pltpu.sync_copy(tidx_hbm.at[pl.ds(base, ROWS_PER_W)], idx_v)
    t0 = lax.gather(
        t0_v[...],
        jnp.zeros((LANES, 1), jnp.int32),
        dimension_numbers=lax.GatherDimensionNumbers(
            offset_dims=(), collapsed_slice_dims=(0,), start_index_map=(0,)),
        slice_sizes=(1,),
        mode=lax.GatherScatterMode.PROMISE_IN_BOUNDS)
    for j in range(ROWS_PER_W // LANES):
        sl = pl.ds(j * LANES, LANES)
        idx_v[sl] = idx_v[sl] - t0

    def gather(ci):
        return pltpu.async_copy(
            pos_hbm.at[idx_v.at[pl.ds(ci * K, K)]], buf_v.at[ci % 2], sem_g)

    def put(ci):
        return pltpu.async_copy(
            buf_v.at[ci % 2], rows_hbm.at[pl.ds(base + ci * K, K)], sem_o)

    g = {0: gather(0)}
    o = {}
    for ci in range(NCHUNK):
        g[ci].wait()
        o[ci] = put(ci)
        if ci >= 1:
            o[ci - 1].wait()
        if ci + 1 < NCHUNK:
            g[ci + 1] = gather(ci + 1)
    o[NCHUNK - 1].wait()


def _tc_add(rows_ref, enc_ref, out_ref):
    out_ref[...] = enc_ref[...] + rows_ref[...][None]


@jax.jit
def _run(enc_inputs, tindex, pos_table):
    mesh = plsc.VectorSubcoreMesh(core_axis_name="c", subcore_axis_name="s")
    gfn = functools.partial(
        pl.kernel,
        mesh=mesh,
        out_type=jax.ShapeDtypeStruct((S, D), jnp.float32),
        scratch_types=[
            pltpu.VMEM((ROWS_PER_W,), jnp.int32),
            pltpu.VMEM((LANES,), jnp.int32),
            pltpu.VMEM((2, K, D), jnp.float32),
            pltpu.SemaphoreType.DMA,
            pltpu.SemaphoreType.DMA,
        ],
    )(_sc_gather)
    rows = gfn(tindex, pos_table)

    add = pl.pallas_call(
        _tc_add,
        grid=(NSB, B),
        in_specs=[
            pl.BlockSpec((BS, D), lambda s, b: (s, 0)),
            pl.BlockSpec((1, BS, D), lambda s, b: (b, s, 0)),
        ],
        out_specs=pl.BlockSpec((1, BS, D), lambda s, b: (b, s, 0)),
        out_shape=jax.ShapeDtypeStruct((B, S, D), jnp.float32),
    )
    return add(rows, enc_inputs)


def kernel(enc_inputs, tindex, pos_table):
    return _run(enc_inputs, tindex, pos_table)


# speedup vs baseline: 1.7346x; 1.1625x over previous
import functools

import jax
import jax.numpy as jnp
from jax import lax
from jax.experimental import pallas as pl
from jax.experimental.pallas import tpu as pltpu
from jax.experimental.pallas import tpu_sc as plsc

B = 4
S = 8192
D = 768
LANES = 16
NC = 2
NS = 16
NW = NC * NS
ROWS_PER_W = S // NW
K = 64
NCHUNK = ROWS_PER_W // K

BS = 512
NSB = S // BS


def _sc_gather(tidx_hbm, pos_hbm, rows_hbm, idx_v, t0_v, buf_v, sem_g,
               sem_o):
    wid = lax.axis_index("s") * NC + lax.axis_index("c")
    base = wid * ROWS_PER_W

    pltpu.sync_copy(tidx_hbm.at[pl.ds(0, LANES)], t0_v)
    pltpu.sync_copy(tidx_hbm.at[pl.ds(base, ROWS_PER_W)], idx_v)
    t0 = lax.gather(
        t0_v[...],
        jnp.zeros((LANES, 1), jnp.int32),
        dimension_numbers=lax.GatherDimensionNumbers(
            offset_dims=(), collapsed_slice_dims=(0,), start_index_map=(0,)),
        slice_sizes=(1,),
        mode=lax.GatherScatterMode.PROMISE_IN_BOUNDS)
    for j in range(ROWS_PER_W // LANES):
        sl = pl.ds(j * LANES, LANES)
        idx_v[sl] = idx_v[sl] - t0

    def gather(ci):
        return pltpu.async_copy(
            pos_hbm.at[idx_v.at[pl.ds(ci * K, K)]], buf_v.at[ci % 2], sem_g)

    def put(ci):
        return pltpu.async_copy(
            buf_v.at[ci % 2], rows_hbm.at[pl.ds(base + ci * K, K)], sem_o)

    g = {0: gather(0)}
    o = {}
    for ci in range(NCHUNK):
        g[ci].wait()
        o[ci] = put(ci)
        if ci >= 1:
            o[ci - 1].wait()
        if ci + 1 < NCHUNK:
            g[ci + 1] = gather(ci + 1)
    o[NCHUNK - 1].wait()


def _tc_add(rows_ref, enc_ref, out_ref):
    out_ref[...] = enc_ref[...] + rows_ref[...][None]


def _tc_add_call(rows, enc_inputs):
    return pl.pallas_call(
        _tc_add,
        grid=(NSB,),
        in_specs=[
            pl.BlockSpec((BS, D), lambda s: (s, 0)),
            pl.BlockSpec((B, BS, D), lambda s: (0, s, 0)),
        ],
        out_specs=pl.BlockSpec((B, BS, D), lambda s: (0, s, 0)),
        out_shape=jax.ShapeDtypeStruct((B, S, D), jnp.float32),
    )(rows, enc_inputs)


@jax.jit
def _run(enc_inputs, tindex, pos_table):
    mesh = plsc.VectorSubcoreMesh(core_axis_name="c", subcore_axis_name="s")
    gfn = functools.partial(
        pl.kernel,
        mesh=mesh,
        out_type=jax.ShapeDtypeStruct((S, D), jnp.float32),
        scratch_types=[
            pltpu.VMEM((ROWS_PER_W,), jnp.int32),
            pltpu.VMEM((LANES,), jnp.int32),
            pltpu.VMEM((2, K, D), jnp.float32),
            pltpu.SemaphoreType.DMA,
            pltpu.SemaphoreType.DMA,
        ],
    )(_sc_gather)
    rows = gfn(tindex, pos_table)
    return _tc_add_call(rows, enc_inputs)


def kernel(enc_inputs, tindex, pos_table):
    return _run(enc_inputs, tindex, pos_table)
